# baseline (speedup 1.0000x reference)
import functools

import jax
import jax.numpy as jnp
from jax import lax
from jax.experimental import pallas as pl
from jax.experimental.pallas import tpu as pltpu
from jax.experimental.pallas import tpu_sc as plsc

B, S, F = 4, 8192, 2048
NC, NS, L = 2, 16, 16
NW = NC * NS
WPB = NW // B
FPW = F // WPB
NLANES = FPW // L
R = 64
NCH = S // R
NBUF = 4
PD = NBUF - 1
NT = NCH // NBUF
assert NCH % NBUF == 0


def _sc_cumsum(x):
    mesh = plsc.VectorSubcoreMesh(core_axis_name="c", subcore_axis_name="s")

    @functools.partial(
        pl.kernel,
        mesh=mesh,
        out_type=jax.ShapeDtypeStruct((B, S, F), jnp.float32),
        scratch_types=[
            pltpu.VMEM((R, FPW), jnp.float32),
            pltpu.VMEM((R, FPW), jnp.float32),
            pltpu.VMEM((R, FPW), jnp.float32),
            pltpu.VMEM((R, FPW), jnp.float32),
            pltpu.SemaphoreType.DMA,
            pltpu.SemaphoreType.DMA,
        ],
    )
    def k(x_hbm, out_hbm, buf0, buf1, buf2, buf3, sem_in, sem_out):
        bufs = (buf0, buf1, buf2, buf3)
        wid = lax.axis_index("s") * NC + lax.axis_index("c")
        b = wid // WPB
        f0 = (wid % WPB) * FPW

        def src(kk):
            return x_hbm.at[b, pl.ds(kk * R, R), pl.ds(f0, FPW)]

        def dst(kk):
            return out_hbm.at[b, pl.ds(kk * R, R), pl.ds(f0, FPW)]

        def start_in(kk, buf):
            pltpu.make_async_copy(src(kk), buf, sem_in).start()

        def wait_in(buf):
            pltpu.make_async_copy(src(0), buf, sem_in).wait()

        def start_out(kk, buf):
            pltpu.make_async_copy(buf, dst(kk), sem_out).start()

        def wait_out(buf):
            pltpu.make_async_copy(buf, dst(0), sem_out).wait()

        def compute(buf, runs):
            def row_body(r, rs):
                new = []
                for c in range(NLANES):
                    sl = pl.ds(c * L, L)
                    v = rs[c] + buf[r, sl]
                    buf[r, sl] = v
                    new.append(v)
                return tuple(new)

            return lax.fori_loop(0, R, row_body, runs, unroll=2)

        runs0 = tuple(jnp.zeros((L,), jnp.float32) for _ in range(NLANES))
        for j in range(PD):
            start_in(j, bufs[j])

        def ring(t, runs):
            k0 = NBUF * t
            for j in range(NBUF):
                kk = k0 + j
                buf = bufs[j]
                wait_in(buf)
                runs = compute(buf, runs)
                start_out(kk, buf)
                nxt = bufs[(j + PD) % NBUF]

                @pl.when(kk + PD < NCH)
                def _():
                    @pl.when(kk >= 1)
                    def _():
                        wait_out(nxt)

                    start_in(kk + PD, nxt)

            return runs

        lax.fori_loop(0, NT, ring, runs0)
        for j in range(NBUF):
            wait_out(bufs[(j + 1) % NBUF])

    return k(x)


def kernel(x, dim, dtype):
    return _sc_cumsum(x)
